# Initial kernel scaffold; baseline (speedup 1.0000x reference)
import jax
import jax.numpy as jnp
from jax.experimental import pallas as pl


def kernel(node_attr, edge_index, edge_attr, edge_world_index, edge_world_attr, W, b):
    raise NotImplementedError("write your pallas kernel here")


# speedup vs baseline: 1.8146x; 1.8146x over previous
import functools

import jax
import jax.numpy as jnp
from jax import lax
from jax.experimental import pallas as pl
from jax.experimental.pallas import tpu as pltpu
from jax.experimental.pallas import tpu_sc as plsc

N_NODES = 10000
N_EDGES = 320000
D = 128
D_EDGE = 16

NC, NS = 2, 16
NW = NC * NS
RPG = 128
K_SUB = 2
CHUNK = K_SUB * RPG
IDX_ROWS = 80
ITERS = IDX_ROWS // K_SUB
PER_W = IDX_ROWS * RPG
N_PAD = PER_W * NW


def _project_body(x_ref, w1_ref, w2_ref, p1_ref, p2_ref):
    x = x_ref[...]
    p1_ref[...] = jnp.dot(x, w1_ref[...], preferred_element_type=jnp.float32)
    p2_ref[...] = jnp.dot(x, w2_ref[...], preferred_element_type=jnp.float32)


def _project(node_attr, w1, w2):
    blk = 1000
    grid = N_NODES // blk
    return pl.pallas_call(
        _project_body,
        grid=(grid,),
        in_specs=[
            pl.BlockSpec((blk, D), lambda i: (i, 0)),
            pl.BlockSpec((D, D), lambda i: (0, 0)),
            pl.BlockSpec((D, D), lambda i: (0, 0)),
        ],
        out_specs=[
            pl.BlockSpec((blk, D), lambda i: (i, 0)),
            pl.BlockSpec((blk, D), lambda i: (i, 0)),
        ],
        out_shape=[
            jax.ShapeDtypeStruct((N_NODES, D), jnp.float32),
            jax.ShapeDtypeStruct((N_NODES, D), jnp.float32),
        ],
    )(node_attr, w1, w2)


def _gather_sum_body(p1_hbm, p2_hbm, sidx_hbm, ridx_hbm, g_hbm,
                     idxs_v, idxr_v, buf1, buf2, sem):
    wid = lax.axis_index("s") * NC + lax.axis_index("c")
    row0 = wid * IDX_ROWS
    pltpu.sync_copy(sidx_hbm.at[pl.ds(row0, IDX_ROWS)], idxs_v)
    pltpu.sync_copy(ridx_hbm.at[pl.ds(row0, IDX_ROWS)], idxr_v)

    def chunk_body(i, carry):
        handles = []
        for j in range(K_SUB):
            dst = pl.ds(j * RPG, RPG)
            handles.append(
                pltpu.async_copy(p1_hbm.at[idxs_v.at[i * K_SUB + j]],
                                 buf1.at[dst], sem))
            handles.append(
                pltpu.async_copy(p2_hbm.at[idxr_v.at[i * K_SUB + j]],
                                 buf2.at[dst], sem))
        for h in handles:
            h.wait()

        def add_row(rw, c):
            for j in range(D // 16):
                sl = pl.ds(j * 16, 16)
                buf1[rw, sl] = buf1[rw, sl] + buf2[rw, sl]
            return c

        lax.fori_loop(0, CHUNK, add_row, 0)
        pltpu.sync_copy(buf1, g_hbm.at[pl.ds(wid * PER_W + i * CHUNK, CHUNK)])
        return carry

    lax.fori_loop(0, ITERS, chunk_body, 0)


def _gather_sum(p1, p2, sidx, ridx):
    mesh = plsc.VectorSubcoreMesh(core_axis_name="c", subcore_axis_name="s",
                                  num_cores=NC, num_subcores=NS)
    kern = pl.kernel(
        _gather_sum_body,
        out_type=jax.ShapeDtypeStruct((N_PAD, D), jnp.float32),
        mesh=mesh,
        scratch_types=[
            pltpu.VMEM((IDX_ROWS, RPG), jnp.int32),
            pltpu.VMEM((IDX_ROWS, RPG), jnp.int32),
            pltpu.VMEM((CHUNK, D), jnp.float32),
            pltpu.VMEM((CHUNK, D), jnp.float32),
            pltpu.SemaphoreType.DMA,
        ],
    )
    return kern(p1, p2, sidx, ridx)


def _epilogue_body(g_ref, e_ref, w3_ref, b_ref, o_ref):
    o_ref[...] = (g_ref[...]
                  + jnp.dot(e_ref[...], w3_ref[...],
                            preferred_element_type=jnp.float32)
                  + b_ref[...])


def _epilogue(g, edge_attr, w3, b2d):
    blk = 2000
    grid = N_EDGES // blk
    return pl.pallas_call(
        _epilogue_body,
        grid=(grid,),
        in_specs=[
            pl.BlockSpec((blk, D), lambda i: (i, 0)),
            pl.BlockSpec((blk, D_EDGE), lambda i: (i, 0)),
            pl.BlockSpec((D_EDGE, D), lambda i: (0, 0)),
            pl.BlockSpec((1, D), lambda i: (0, 0)),
        ],
        out_specs=pl.BlockSpec((blk, D), lambda i: (i, 0)),
        out_shape=jax.ShapeDtypeStruct((N_EDGES, D), jnp.float32),
    )(g, edge_attr, w3, b2d)


def kernel(node_attr, edge_index, edge_attr, edge_world_index, edge_world_attr, W, b):
    w1 = W[:D]
    w2 = W[D:2 * D]
    w3 = W[2 * D:]
    b2d = b.reshape(1, D)

    p1, p2 = _project(node_attr, w1, w2)

    pad = N_PAD - N_EDGES
    sidx = jnp.pad(edge_index[0], (0, pad)).reshape(N_PAD // RPG, RPG)
    ridx = jnp.pad(edge_index[1], (0, pad)).reshape(N_PAD // RPG, RPG)

    g = _gather_sum(p1, p2, sidx, ridx)

    edge_attr_ = _epilogue(g, edge_attr, w3, b2d)
    return (node_attr, edge_attr_, edge_index, edge_world_index, edge_world_attr)


# speedup vs baseline: 2.0939x; 1.1539x over previous
import functools

import jax
import jax.numpy as jnp
from jax import lax
from jax.experimental import pallas as pl
from jax.experimental.pallas import tpu as pltpu
from jax.experimental.pallas import tpu_sc as plsc

N_NODES = 10000
N_EDGES = 320000
D = 128
D_EDGE = 16

NC, NS = 2, 16
NW = NC * NS
RPG = 128
CHUNK = RPG
IDX_ROWS = 80
BATCHES = IDX_ROWS
PER_W = IDX_ROWS * RPG
N_PAD = PER_W * NW


def _project_body(x_ref, w1_ref, w2_ref, p1_ref, p2_ref):
    x = x_ref[...]
    p1_ref[...] = jnp.dot(x, w1_ref[...], preferred_element_type=jnp.float32)
    p2_ref[...] = jnp.dot(x, w2_ref[...], preferred_element_type=jnp.float32)


def _project(node_attr, w1, w2):
    blk = 1000
    grid = N_NODES // blk
    return pl.pallas_call(
        _project_body,
        grid=(grid,),
        in_specs=[
            pl.BlockSpec((blk, D), lambda i: (i, 0)),
            pl.BlockSpec((D, D), lambda i: (0, 0)),
            pl.BlockSpec((D, D), lambda i: (0, 0)),
        ],
        out_specs=[
            pl.BlockSpec((blk, D), lambda i: (i, 0)),
            pl.BlockSpec((blk, D), lambda i: (i, 0)),
        ],
        out_shape=[
            jax.ShapeDtypeStruct((N_NODES, D), jnp.float32),
            jax.ShapeDtypeStruct((N_NODES, D), jnp.float32),
        ],
    )(node_attr, w1, w2)


def _gather_sum_body(p1_hbm, p2_hbm, sidx_hbm, ridx_hbm, g_hbm,
                     idxs_v, idxr_v, g1a, g2a, g1b, g2b, oa, ob,
                     gsa, gsb, wsa, wsb):
    wid = lax.axis_index("s") * NC + lax.axis_index("c")
    row0 = wid * IDX_ROWS
    pltpu.sync_copy(sidx_hbm.at[pl.ds(row0, IDX_ROWS)], idxs_v)
    pltpu.sync_copy(ridx_hbm.at[pl.ds(row0, IDX_ROWS)], idxr_v)

    def issue_gather(b, g1, g2, gs):
        pltpu.async_copy(p1_hbm.at[idxs_v.at[b]], g1, gs)
        pltpu.async_copy(p2_hbm.at[idxr_v.at[b]], g2, gs)

    slots = ((g1a, g2a, oa, gsa, wsa), (g1b, g2b, ob, gsb, wsb))

    issue_gather(0, g1a, g2a, gsa)
    issue_gather(1, g1b, g2b, gsb)

    def outer(it, carry):
        for sl_i in range(2):
            g1, g2, o, gs, ws = slots[sl_i]
            bi = it * 2 + sl_i
            pltpu.make_async_copy(p1_hbm.at[pl.ds(0, RPG)], g1, gs).wait()
            pltpu.make_async_copy(p1_hbm.at[pl.ds(0, RPG)], g2, gs).wait()

            @pl.when(bi >= 2)
            def _():
                pltpu.make_async_copy(o, g_hbm.at[pl.ds(0, CHUNK)], ws).wait()

            def add_row(rw, c):
                for j in range(D // 16):
                    s2 = pl.ds(j * 16, 16)
                    o[rw, s2] = g1[rw, s2] + g2[rw, s2]
                return c

            lax.fori_loop(0, CHUNK, add_row, 0)
            pltpu.async_copy(
                o, g_hbm.at[pl.ds(wid * PER_W + bi * CHUNK, CHUNK)], ws)

            @pl.when(bi + 2 < BATCHES)
            def _():
                issue_gather(bi + 2, g1, g2, gs)
        return carry

    lax.fori_loop(0, BATCHES // 2, outer, 0)

    pltpu.make_async_copy(oa, g_hbm.at[pl.ds(0, CHUNK)], wsa).wait()
    pltpu.make_async_copy(ob, g_hbm.at[pl.ds(0, CHUNK)], wsb).wait()


def _gather_sum(p1, p2, sidx, ridx):
    mesh = plsc.VectorSubcoreMesh(core_axis_name="c", subcore_axis_name="s",
                                  num_cores=NC, num_subcores=NS)
    kern = pl.kernel(
        _gather_sum_body,
        out_type=jax.ShapeDtypeStruct((N_PAD, D), jnp.float32),
        mesh=mesh,
        scratch_types=[
            pltpu.VMEM((IDX_ROWS, RPG), jnp.int32),
            pltpu.VMEM((IDX_ROWS, RPG), jnp.int32),
            pltpu.VMEM((CHUNK, D), jnp.float32),
            pltpu.VMEM((CHUNK, D), jnp.float32),
            pltpu.VMEM((CHUNK, D), jnp.float32),
            pltpu.VMEM((CHUNK, D), jnp.float32),
            pltpu.VMEM((CHUNK, D), jnp.float32),
            pltpu.VMEM((CHUNK, D), jnp.float32),
            pltpu.SemaphoreType.DMA,
            pltpu.SemaphoreType.DMA,
            pltpu.SemaphoreType.DMA,
            pltpu.SemaphoreType.DMA,
        ],
    )
    return kern(p1, p2, sidx, ridx)


def _epilogue_body(g_ref, e_ref, w3_ref, b_ref, o_ref):
    o_ref[...] = (g_ref[...]
                  + jnp.dot(e_ref[...], w3_ref[...],
                            preferred_element_type=jnp.float32)
                  + b_ref[...])


def _epilogue(g, edge_attr, w3, b2d):
    blk = 2000
    grid = N_EDGES // blk
    return pl.pallas_call(
        _epilogue_body,
        grid=(grid,),
        in_specs=[
            pl.BlockSpec((blk, D), lambda i: (i, 0)),
            pl.BlockSpec((blk, D_EDGE), lambda i: (i, 0)),
            pl.BlockSpec((D_EDGE, D), lambda i: (0, 0)),
            pl.BlockSpec((1, D), lambda i: (0, 0)),
        ],
        out_specs=pl.BlockSpec((blk, D), lambda i: (i, 0)),
        out_shape=jax.ShapeDtypeStruct((N_EDGES, D), jnp.float32),
    )(g, edge_attr, w3, b2d)


def kernel(node_attr, edge_index, edge_attr, edge_world_index, edge_world_attr, W, b):
    w1 = W[:D]
    w2 = W[D:2 * D]
    w3 = W[2 * D:]
    b2d = b.reshape(1, D)

    p1, p2 = _project(node_attr, w1, w2)

    pad = N_PAD - N_EDGES
    sidx = jnp.pad(edge_index[0], (0, pad)).reshape(N_PAD // RPG, RPG)
    ridx = jnp.pad(edge_index[1], (0, pad)).reshape(N_PAD // RPG, RPG)

    g = _gather_sum(p1, p2, sidx, ridx)

    edge_attr_ = _epilogue(g, edge_attr, w3, b2d)
    return (node_attr, edge_attr_, edge_index, edge_world_index, edge_world_attr)
